# baseline (speedup 1.0000x reference)
import functools

import jax
import jax.numpy as jnp
from jax import lax
from jax.experimental import pallas as pl
from jax.experimental.pallas import tpu as pltpu
from jax.experimental.pallas import tpu_sc as plsc

L = 200
D = 64
NB = 4096
LANES = 16
NC = 2
NS = 16
NW = NC * NS
CHUNK = 256
NCHUNK = NB // CHUNK
MAXL_W = (L + NW - 1) // NW
NBUF = 4


@functools.cache
def _build():
    mesh = plsc.VectorSubcoreMesh(core_axis_name="c", subcore_axis_name="s")

    @functools.partial(
        pl.kernel,
        mesh=mesh,
        out_type=jax.ShapeDtypeStruct((L, D, NB), jnp.float32),
        scratch_types=[
            pltpu.VMEM((MAXL_W, NB), jnp.int32),
            pltpu.VMEM((2, D), jnp.float32),
            pltpu.VMEM((NBUF, CHUNK, D), jnp.float32),
            pltpu.VMEM((2, D, CHUNK + 1), jnp.float32),
            pltpu.SemaphoreType.DMA,
            pltpu.SemaphoreType.DMA,
            pltpu.SemaphoreType.DMA,
            pltpu.SemaphoreType.DMA,
            pltpu.SemaphoreType.DMA,
            pltpu.SemaphoreType.DMA,
            pltpu.SemaphoreType.DMA,
        ],
        compiler_params=pltpu.CompilerParams(
            use_tc_tiling_on_sc=False, needs_layout_passes=False,
            skip_device_barrier=True,
        ),
    )
    def run(seq_hbm, tok_hbm, pos_hbm, out_hbm,
            idx_all, pos2, rows_all, xout_all,
            g0, g1, g2, g3, o0, o1, psem):
        gsem = (g0, g1, g2, g3)
        osem = (o0, o1)
        w = lax.axis_index("s") * NC + lax.axis_index("c")
        n_l = (L - 1 - w) // NW + 1
        nj = n_l * NCHUNK

        def stage(i, _):
            pltpu.sync_copy(seq_hbm.at[w + i * NW], idx_all.at[i])
            return 0

        lax.fori_loop(0, n_l, stage, 0)
        pltpu.sync_copy(pos_hbm.at[w], pos2.at[0])

        iota = lax.iota(jnp.int32, LANES)
        row16 = [iota + b0 for b0 in range(0, CHUNK, LANES)]

        def fire_gathers(j, b):
            li = j // NCHUNK
            c = j % NCHUNK
            pltpu.async_copy(
                tok_hbm.at[idx_all.at[li, pl.ds(c * CHUNK, CHUNK)]],
                rows_all.at[b],
                gsem[b],
            )

        def drain(dummy_src, dst, sem):
            pltpu.make_async_copy(dummy_src, dst, sem).wait()

        for j0 in range(NBUF - 1):
            @pl.when(j0 < nj)
            def _():
                fire_gathers(jnp.int32(j0), j0)

        def quad_body(jj, _):
            for b in range(NBUF):
                j = jj * NBUF + b
                li = j // NCHUNK
                c = j % NCHUNK
                l = w + li * NW
                xb = b & 1

                @pl.when(j + NBUF - 1 < nj)
                def _():
                    fire_gathers(j + NBUF - 1, (b + NBUF - 1) % NBUF)

                @pl.when(jnp.logical_and(c == 8, li + 1 < n_l))
                def _():
                    pltpu.async_copy(pos_hbm.at[w + (li + 1) * NW],
                                     pos2.at[(li + 1) & 1], psem)

                drain(tok_hbm.at[pl.ds(0, CHUNK)], rows_all.at[b], gsem[b])

                @pl.when(jnp.logical_and(c == 0, li > 0))
                def _():
                    drain(pos_hbm.at[0], pos2.at[0], psem)

                @pl.when(j >= 2)
                def _():
                    drain(out_hbm.at[0, :, pl.ds(0, CHUNK)],
                          xout_all.at[xb, :, pl.ds(0, CHUNK)], osem[xb])

                pcol = [pos2[li & 1, pl.ds(d0, LANES)]
                        for d0 in range(0, D, LANES)]

                def r_body(r2, _):
                    r = r2 * 2
                    vs = []
                    for u in range(2):
                        colv = jnp.full((LANES,), r + u, jnp.int32)
                        for k in range(D // LANES):
                            vs.append((
                                rows_all[b, r + u, pl.ds(k * LANES, LANES)]
                                + pcol[k],
                                row16[k], colv))
                    for v, rowv, colv in vs:
                        plsc.store_scatter(xout_all.at[xb], [rowv, colv], v)
                    return 0

                lax.fori_loop(0, CHUNK // 2, r_body, 0)

                pltpu.async_copy(
                    xout_all.at[xb, :, pl.ds(0, CHUNK)],
                    out_hbm.at[l, :, pl.ds(c * CHUNK, CHUNK)],
                    osem[xb],
                )
            return 0

        lax.fori_loop(0, nj // NBUF, quad_body, 0)

        for xb in range(2):
            drain(out_hbm.at[0, :, pl.ds(0, CHUNK)],
                  xout_all.at[xb, :, pl.ds(0, CHUNK)], osem[xb])

    return run


def kernel(seq, token_table, pos_table):
    b, l = seq.shape
    d = token_table.shape[1]
    seq3 = seq.T
    out = _build()(seq3, token_table, pos_table)
    return jnp.transpose(out, (2, 0, 1))


# speedup vs baseline: 1.0047x; 1.0047x over previous
import functools

import jax
import jax.numpy as jnp
from jax import lax
from jax.experimental import pallas as pl
from jax.experimental.pallas import tpu as pltpu
from jax.experimental.pallas import tpu_sc as plsc

L = 200
D = 64
NB = 4096
LANES = 16
NC = 2
NS = 16
NW = NC * NS
CHUNK = 256
NCHUNK = NB // CHUNK
NBUF = 5


@functools.cache
def _build():
    mesh = plsc.VectorSubcoreMesh(core_axis_name="c", subcore_axis_name="s")

    @functools.partial(
        pl.kernel,
        mesh=mesh,
        out_type=jax.ShapeDtypeStruct((L, D, NB), jnp.float32),
        scratch_types=[
            pltpu.VMEM((2, NB), jnp.int32),
            pltpu.VMEM((2, D), jnp.float32),
            pltpu.VMEM((NBUF, CHUNK, D), jnp.float32),
            pltpu.VMEM((2, D, CHUNK + 1), jnp.float32),
            pltpu.SemaphoreType.DMA((NBUF,)),
            pltpu.SemaphoreType.DMA((2,)),
            pltpu.SemaphoreType.DMA,
        ],
        compiler_params=pltpu.CompilerParams(
            use_tc_tiling_on_sc=False, needs_layout_passes=False,
        ),
    )
    def run(seq_hbm, tok_hbm, pos_hbm, out_hbm,
            idx2, pos2, rows_all, xout_all, gsem, osem, isem):
        w = lax.axis_index("s") * NC + lax.axis_index("c")
        n_l = (L - 1 - w) // NW + 1
        nj = n_l * NCHUNK

        pltpu.sync_copy(seq_hbm.at[w], idx2.at[0])
        pltpu.sync_copy(pos_hbm.at[w], pos2.at[0])

        iota = lax.iota(jnp.int32, LANES)
        row16 = [iota + d0 for d0 in range(0, D, LANES)]

        def fire_gather(j):
            li = j // NCHUNK
            c = j % NCHUNK
            slot = j % NBUF
            pltpu.async_copy(
                tok_hbm.at[idx2.at[li & 1, pl.ds(c * CHUNK, CHUNK)]],
                rows_all.at[slot],
                gsem.at[slot],
            )

        def drain(dummy_src, dst, sem):
            pltpu.make_async_copy(dummy_src, dst, sem).wait()

        for j0 in range(NBUF - 1):
            @pl.when(j0 < nj)
            def _():
                fire_gather(jnp.int32(j0))

        def body(j, _):
            li = j // NCHUNK
            c = j % NCHUNK
            slot = j % NBUF
            xb = j & 1
            l = w + li * NW

            @pl.when(jnp.logical_and(c == 8, li + 1 < n_l))
            def _():
                nb = (li + 1) & 1
                pltpu.async_copy(seq_hbm.at[w + (li + 1) * NW],
                                 idx2.at[nb], isem)
                pltpu.async_copy(pos_hbm.at[w + (li + 1) * NW],
                                 pos2.at[nb], isem)

            @pl.when(jnp.logical_and(c == NCHUNK - (NBUF - 1),
                                     li + 1 < n_l))
            def _():
                drain(seq_hbm.at[0], idx2.at[0], isem)
                drain(pos_hbm.at[0], pos2.at[0], isem)

            @pl.when(j + NBUF - 1 < nj)
            def _():
                fire_gather(j + NBUF - 1)

            drain(tok_hbm.at[pl.ds(0, CHUNK)], rows_all.at[slot],
                  gsem.at[slot])

            @pl.when(j >= 2)
            def _():
                drain(out_hbm.at[0, :, pl.ds(0, CHUNK)],
                      xout_all.at[xb, :, pl.ds(0, CHUNK)], osem.at[xb])

            pcol = [pos2[li & 1, pl.ds(d0, LANES)]
                    for d0 in range(0, D, LANES)]

            def r_body(r2, _):
                r = r2 * 4
                vs = []
                for u in range(4):
                    colv = jnp.full((LANES,), r + u, jnp.int32)
                    for k in range(D // LANES):
                        vs.append((
                            rows_all[slot, r + u, pl.ds(k * LANES, LANES)]
                            + pcol[k],
                            row16[k], colv))
                for v, rowv, colv in vs:
                    plsc.store_scatter(xout_all.at[xb], [rowv, colv], v)
                return 0

            lax.fori_loop(0, CHUNK // 4, r_body, 0)

            pltpu.async_copy(
                xout_all.at[xb, :, pl.ds(0, CHUNK)],
                out_hbm.at[l, :, pl.ds(c * CHUNK, CHUNK)],
                osem.at[xb],
            )
            return 0

        lax.fori_loop(0, nj, body, 0)

        for xb in range(2):
            drain(out_hbm.at[0, :, pl.ds(0, CHUNK)],
                  xout_all.at[xb, :, pl.ds(0, CHUNK)], osem.at[xb])

    return run


def kernel(seq, token_table, pos_table):
    out = _build()(seq.T, token_table, pos_table)
    return jnp.transpose(out, (2, 0, 1))


# speedup vs baseline: 1.0143x; 1.0095x over previous
import functools

import jax
import jax.numpy as jnp
from jax import lax
from jax.experimental import pallas as pl
from jax.experimental.pallas import tpu as pltpu
from jax.experimental.pallas import tpu_sc as plsc

L = 200
D = 64
NB = 4096
LANES = 16
NC = 2
NS = 16
NW = NC * NS
CHUNK = 256
NCHUNK = NB // CHUNK
NBUF = 3
WPAIR = 2


@functools.cache
def _build():
    mesh = plsc.VectorSubcoreMesh(core_axis_name="c", subcore_axis_name="s")

    @functools.partial(
        pl.kernel,
        mesh=mesh,
        out_type=jax.ShapeDtypeStruct((L, D, NB), jnp.float32),
        scratch_types=[
            pltpu.VMEM((2, NB), jnp.int32),
            pltpu.VMEM((2, D), jnp.float32),
            pltpu.VMEM((NBUF, CHUNK, D), jnp.float32),
            pltpu.VMEM((2, D, WPAIR * CHUNK + 1), jnp.float32),
            pltpu.SemaphoreType.DMA((NBUF,)),
            pltpu.SemaphoreType.DMA((2,)),
            pltpu.SemaphoreType.DMA,
        ],
        compiler_params=pltpu.CompilerParams(
            use_tc_tiling_on_sc=False, needs_layout_passes=False,
        ),
    )
    def run(seq_hbm, tok_hbm, pos_hbm, out_hbm,
            idx2, pos2, rows_all, xout_all, gsem, osem, isem):
        w = lax.axis_index("s") * NC + lax.axis_index("c")
        n_l = (L - 1 - w) // NW + 1
        nj = n_l * NCHUNK

        pltpu.sync_copy(seq_hbm.at[w], idx2.at[0])
        pltpu.sync_copy(pos_hbm.at[w], pos2.at[0])

        iota = lax.iota(jnp.int32, LANES)
        row16 = [iota + d0 for d0 in range(0, D, LANES)]

        def fire_gather(j):
            li = j // NCHUNK
            c = j % NCHUNK
            slot = j % NBUF
            pltpu.async_copy(
                tok_hbm.at[idx2.at[li & 1, pl.ds(c * CHUNK, CHUNK)]],
                rows_all.at[slot],
                gsem.at[slot],
            )

        def drain(dummy_src, dst, sem):
            pltpu.make_async_copy(dummy_src, dst, sem).wait()

        for j0 in range(NBUF - 1):
            @pl.when(j0 < nj)
            def _():
                fire_gather(jnp.int32(j0))

        def body(j, _):
            li = j // NCHUNK
            c = j % NCHUNK
            slot = j % NBUF
            xb = (j >> 1) & 1
            l = w + li * NW

            @pl.when(jnp.logical_and(c == 8, li + 1 < n_l))
            def _():
                nb = (li + 1) & 1
                pltpu.async_copy(seq_hbm.at[w + (li + 1) * NW],
                                 idx2.at[nb], isem)
                pltpu.async_copy(pos_hbm.at[w + (li + 1) * NW],
                                 pos2.at[nb], isem)

            @pl.when(jnp.logical_and(c == NCHUNK - (NBUF - 1),
                                     li + 1 < n_l))
            def _():
                drain(seq_hbm.at[0], idx2.at[0], isem)
                drain(pos_hbm.at[0], pos2.at[0], isem)

            @pl.when(j + NBUF - 1 < nj)
            def _():
                fire_gather(j + NBUF - 1)

            drain(tok_hbm.at[pl.ds(0, CHUNK)], rows_all.at[slot],
                  gsem.at[slot])

            @pl.when(jnp.logical_and((j & 1) == 0, (j >> 1) >= 2))
            def _():
                drain(out_hbm.at[0, :, pl.ds(0, WPAIR * CHUNK)],
                      xout_all.at[xb, :, pl.ds(0, WPAIR * CHUNK)],
                      osem.at[xb])

            pcol = [pos2[li & 1, pl.ds(d0, LANES)]
                    for d0 in range(0, D, LANES)]

            cbase = (c & 1) * CHUNK

            def r_body(r2, _):
                r = r2 * 4
                vs = []
                for u in range(4):
                    colv = jnp.full((LANES,), cbase + r + u, jnp.int32)
                    for k in range(D // LANES):
                        vs.append((
                            rows_all[slot, r + u, pl.ds(k * LANES, LANES)]
                            + pcol[k],
                            row16[k], colv))
                for v, rowv, colv in vs:
                    plsc.store_scatter(xout_all.at[xb], [rowv, colv], v)
                return 0

            lax.fori_loop(0, CHUNK // 4, r_body, 0)

            @pl.when((j & 1) == 1)
            def _():
                pltpu.async_copy(
                    xout_all.at[xb, :, pl.ds(0, WPAIR * CHUNK)],
                    out_hbm.at[l, :, pl.ds((c - 1) * CHUNK, WPAIR * CHUNK)],
                    osem.at[xb],
                )
            return 0

        lax.fori_loop(0, nj, body, 0)

        for xb in range(2):
            drain(out_hbm.at[0, :, pl.ds(0, WPAIR * CHUNK)],
                  xout_all.at[xb, :, pl.ds(0, WPAIR * CHUNK)], osem.at[xb])

    return run


def kernel(seq, token_table, pos_table):
    out = _build()(seq.T, token_table, pos_table)
    return jnp.transpose(out, (2, 0, 1))
